# Initial kernel scaffold; baseline (speedup 1.0000x reference)
import jax
import jax.numpy as jnp
from jax.experimental import pallas as pl


def kernel(x, adj, W1, b1, W2, b2, W3, b3, W4, b4, W5, b5, W6, b6, W7, b7, W8, b8):
    raise NotImplementedError("write your pallas kernel here")


# speedup vs baseline: 1.5758x; 1.5758x over previous
import functools

import jax
import jax.numpy as jnp
from jax.experimental import pallas as pl

_QSCALE = 254.0
_QOFF = 127.0


def _support0_kernel(x_ref, w_ref, o_ref):
    o_ref[...] = jnp.dot(
        x_ref[...], w_ref[...], preferred_element_type=jnp.float32
    ).astype(jnp.bfloat16)


def _layer1_kernel(adj_ref, s_ref, b_ref, wn_ref, q_ref, sn_ref):
    a = adj_ref[...]
    q_ref[...] = (jnp.round(a * _QSCALE) - _QOFF).astype(jnp.int8)
    h = jnp.dot(a.astype(jnp.bfloat16), s_ref[...],
                preferred_element_type=jnp.float32)
    h = jnp.maximum(h + b_ref[...], 0.0)
    sn_ref[...] = jnp.dot(h, wn_ref[...],
                          preferred_element_type=jnp.float32).astype(jnp.bfloat16)


def _mid_kernel(q_ref, s_ref, b_ref, wn_ref, sn_ref):
    s = s_ref[...]
    corr = jnp.sum(s.astype(jnp.float32), axis=0, keepdims=True)
    acc = jnp.dot(q_ref[...].astype(jnp.bfloat16), s,
                  preferred_element_type=jnp.float32)
    h = acc * (1.0 / _QSCALE) + corr * (_QOFF / _QSCALE) + b_ref[...]
    h = jnp.maximum(h, 0.0)
    sn_ref[...] = jnp.dot(h, wn_ref[...],
                          preferred_element_type=jnp.float32).astype(jnp.bfloat16)


def _last_kernel(q_ref, s_ref, b_ref, o_ref):
    s = s_ref[...]
    corr = jnp.sum(s.astype(jnp.float32), axis=0, keepdims=True)
    acc = jnp.dot(q_ref[...].astype(jnp.bfloat16), s,
                  preferred_element_type=jnp.float32)
    o_ref[...] = acc * (1.0 / _QSCALE) + corr * (_QOFF / _QSCALE) + b_ref[...]


@functools.partial(jax.jit, static_argnames=("bm1", "bm"))
def _gcn8(x, adj, Ws, bs, bm1=400, bm=400):
    n, nfeat = x.shape
    nhid = Ws[0].shape[1]
    nclass = Ws[7].shape[1]
    f32 = jnp.float32

    s = pl.pallas_call(
        _support0_kernel,
        out_shape=jax.ShapeDtypeStruct((n, nhid), jnp.bfloat16),
    )(x, Ws[0])

    m1 = n // bm1
    q, s = pl.pallas_call(
        _layer1_kernel,
        grid=(m1,),
        in_specs=[
            pl.BlockSpec((bm1, n), lambda m: (m, 0)),
            pl.BlockSpec((n, nhid), lambda m: (0, 0)),
            pl.BlockSpec((1, nhid), lambda m: (0, 0)),
            pl.BlockSpec((nhid, nhid), lambda m: (0, 0)),
        ],
        out_specs=[
            pl.BlockSpec((bm1, n), lambda m: (m, 0)),
            pl.BlockSpec((bm1, nhid), lambda m: (m, 0)),
        ],
        out_shape=[
            jax.ShapeDtypeStruct((n, n), jnp.int8),
            jax.ShapeDtypeStruct((n, nhid), jnp.bfloat16),
        ],
    )(adj, s, bs[0].reshape(1, nhid).astype(f32), Ws[1])

    m = n // bm
    for i in range(1, 7):
        s = pl.pallas_call(
            _mid_kernel,
            grid=(m,),
            in_specs=[
                pl.BlockSpec((bm, n), lambda mm: (mm, 0)),
                pl.BlockSpec((n, nhid), lambda mm: (0, 0)),
                pl.BlockSpec((1, nhid), lambda mm: (0, 0)),
                pl.BlockSpec((nhid, Ws[i + 1].shape[1]), lambda mm: (0, 0)),
            ],
            out_specs=pl.BlockSpec((bm, Ws[i + 1].shape[1]), lambda mm: (mm, 0)),
            out_shape=jax.ShapeDtypeStruct((n, Ws[i + 1].shape[1]), jnp.bfloat16),
        )(q, s, bs[i].reshape(1, nhid).astype(f32), Ws[i + 1])

    out = pl.pallas_call(
        _last_kernel,
        grid=(m,),
        in_specs=[
            pl.BlockSpec((bm, n), lambda mm: (mm, 0)),
            pl.BlockSpec((n, nclass), lambda mm: (0, 0)),
            pl.BlockSpec((1, nclass), lambda mm: (0, 0)),
        ],
        out_specs=pl.BlockSpec((bm, nclass), lambda mm: (mm, 0)),
        out_shape=jax.ShapeDtypeStruct((n, nclass), f32),
    )(q, s, bs[7].reshape(1, nclass).astype(f32))
    return out


def kernel(x, adj, W1, b1, W2, b2, W3, b3, W4, b4, W5, b5, W6, b6, W7, b7, W8, b8):
    Ws = (W1, W2, W3, W4, W5, W6, W7, W8)
    bs = (b1, b2, b3, b4, b5, b6, b7, b8)
    return _gcn8(x, adj, Ws, bs)


# speedup vs baseline: 1.7227x; 1.0932x over previous
import functools

import jax
import jax.numpy as jnp
from jax.experimental import pallas as pl

_F8 = jnp.float8_e4m3fn
_F8MAX = 448.0


def _support0_kernel(x_ref, w_ref, o_ref):
    o_ref[...] = jnp.dot(
        x_ref[...], w_ref[...], preferred_element_type=jnp.float32
    ).astype(jnp.bfloat16)


def _layer1_kernel(adj_ref, s_ref, b_ref, wn_ref, a8_ref, rs_ref, sn_ref):
    a = adj_ref[...]
    a8_ref[...] = a.astype(_F8)
    rs_ref[...] = jnp.sum(a, axis=1, keepdims=True)
    h = jnp.dot(a.astype(jnp.bfloat16), s_ref[...],
                preferred_element_type=jnp.float32)
    h = jnp.maximum(h + b_ref[...], 0.0)
    sn_ref[...] = jnp.dot(h, wn_ref[...], preferred_element_type=jnp.float32)


def _prep_kernel(s_ref, t8_ref, m_ref, g_ref):
    s = s_ref[...]
    n = s.shape[0]
    m = jnp.sum(s, axis=0, keepdims=True) * (1.0 / n)
    t = s - m
    g = jnp.maximum(jnp.max(jnp.abs(t), axis=0, keepdims=True), 1e-30) * (
        1.0 / _F8MAX)
    m_ref[...] = m
    g_ref[...] = g
    t8_ref[...] = (t * (1.0 / g)).astype(_F8)


def _mid_kernel(a8_ref, t8_ref, m_ref, g_ref, rs_ref, b_ref, wn_ref, sn_ref):
    acc = jnp.dot(a8_ref[...], t8_ref[...], preferred_element_type=jnp.float32)
    h = acc * g_ref[...] + rs_ref[...] * m_ref[...] + b_ref[...]
    h = jnp.maximum(h, 0.0)
    sn_ref[...] = jnp.dot(h, wn_ref[...], preferred_element_type=jnp.float32)


def _last_kernel(a8_ref, t8_ref, m_ref, g_ref, rs_ref, b_ref, o_ref):
    acc = jnp.dot(a8_ref[...], t8_ref[...], preferred_element_type=jnp.float32)
    o_ref[...] = acc * g_ref[...] + rs_ref[...] * m_ref[...] + b_ref[...]


def _prep(s, nh):
    n = s.shape[0]
    return pl.pallas_call(
        _prep_kernel,
        out_shape=[
            jax.ShapeDtypeStruct((n, nh), _F8),
            jax.ShapeDtypeStruct((1, nh), jnp.float32),
            jax.ShapeDtypeStruct((1, nh), jnp.float32),
        ],
    )(s)


@functools.partial(jax.jit, static_argnames=("bm1", "bm"))
def _gcn8(x, adj, Ws, bs, bm1=400, bm=400):
    n, _ = x.shape
    nhid = Ws[0].shape[1]
    nclass = Ws[7].shape[1]
    f32 = jnp.float32

    s = pl.pallas_call(
        _support0_kernel,
        out_shape=jax.ShapeDtypeStruct((n, nhid), jnp.bfloat16),
    )(x, Ws[0])

    m1 = n // bm1
    a8, rs, s = pl.pallas_call(
        _layer1_kernel,
        grid=(m1,),
        in_specs=[
            pl.BlockSpec((bm1, n), lambda m: (m, 0)),
            pl.BlockSpec((n, nhid), lambda m: (0, 0)),
            pl.BlockSpec((1, nhid), lambda m: (0, 0)),
            pl.BlockSpec((nhid, nhid), lambda m: (0, 0)),
        ],
        out_specs=[
            pl.BlockSpec((bm1, n), lambda m: (m, 0)),
            pl.BlockSpec((bm1, 1), lambda m: (m, 0)),
            pl.BlockSpec((bm1, nhid), lambda m: (m, 0)),
        ],
        out_shape=[
            jax.ShapeDtypeStruct((n, n), _F8),
            jax.ShapeDtypeStruct((n, 1), f32),
            jax.ShapeDtypeStruct((n, nhid), f32),
        ],
    )(adj, s, bs[0].reshape(1, nhid).astype(f32), Ws[1])

    mg = n // bm
    for i in range(1, 7):
        t8, m, g = _prep(s, nhid)
        s = pl.pallas_call(
            _mid_kernel,
            grid=(mg,),
            in_specs=[
                pl.BlockSpec((bm, n), lambda mm: (mm, 0)),
                pl.BlockSpec((n, nhid), lambda mm: (0, 0)),
                pl.BlockSpec((1, nhid), lambda mm: (0, 0)),
                pl.BlockSpec((1, nhid), lambda mm: (0, 0)),
                pl.BlockSpec((bm, 1), lambda mm: (mm, 0)),
                pl.BlockSpec((1, nhid), lambda mm: (0, 0)),
                pl.BlockSpec((nhid, Ws[i + 1].shape[1]), lambda mm: (0, 0)),
            ],
            out_specs=pl.BlockSpec((bm, Ws[i + 1].shape[1]), lambda mm: (mm, 0)),
            out_shape=jax.ShapeDtypeStruct((n, Ws[i + 1].shape[1]), f32),
        )(a8, t8, m, g, rs, bs[i].reshape(1, nhid).astype(f32), Ws[i + 1])

    t8, m, g = _prep(s, nclass)
    out = pl.pallas_call(
        _last_kernel,
        grid=(mg,),
        in_specs=[
            pl.BlockSpec((bm, n), lambda mm: (mm, 0)),
            pl.BlockSpec((n, nclass), lambda mm: (0, 0)),
            pl.BlockSpec((1, nclass), lambda mm: (0, 0)),
            pl.BlockSpec((1, nclass), lambda mm: (0, 0)),
            pl.BlockSpec((bm, 1), lambda mm: (mm, 0)),
            pl.BlockSpec((1, nclass), lambda mm: (0, 0)),
        ],
        out_specs=pl.BlockSpec((bm, nclass), lambda mm: (mm, 0)),
        out_shape=jax.ShapeDtypeStruct((n, nclass), f32),
    )(a8, t8, m, g, rs, bs[7].reshape(1, nclass).astype(f32))
    return out


def kernel(x, adj, W1, b1, W2, b2, W3, b3, W4, b4, W5, b5, W6, b6, W7, b7, W8, b8):
    Ws = (W1, W2, W3, W4, W5, W6, W7, W8)
    bs = (b1, b2, b3, b4, b5, b6, b7, b8)
    return _gcn8(x, adj, Ws, bs)


# speedup vs baseline: 1.9584x; 1.1368x over previous
import functools

import jax
import jax.numpy as jnp
from jax.experimental import pallas as pl
from jax.experimental.pallas import tpu as pltpu

_F8 = jnp.float8_e4m3fn
_F8MAX = 448.0


def _support0_kernel(x_ref, w_ref, o_ref):
    o_ref[...] = jnp.dot(
        x_ref[...], w_ref[...], preferred_element_type=jnp.float32
    ).astype(jnp.bfloat16)


def _layer1_kernel(adj_ref, s_ref, b_ref, wn_ref,
                   a8h_ref, a8c_ref, rs_ref, sn_ref, *, k):
    a = adj_ref[...]
    a8 = a.astype(_F8)
    bm = a8.shape[0]
    a8h_ref[...] = a8[:, :k].reshape(1, bm, k)
    a8c_ref[...] = a8[:, k:]
    rs_ref[...] = jnp.sum(a, axis=1, keepdims=True)
    h = jnp.dot(a.astype(jnp.bfloat16), s_ref[...],
                preferred_element_type=jnp.float32)
    h = jnp.maximum(h + b_ref[...], 0.0)
    sn_ref[...] = jnp.dot(h, wn_ref[...], preferred_element_type=jnp.float32)


def _fused7_kernel(a8h_ref, a8c_ref, rs_ref, s2_ref, w_ref, b_ref,
                   o_ref, s_scr, t8_scr, m_scr, g_scr, *, k, bm, nclass):
    l = pl.program_id(0)
    mm = pl.program_id(1)
    n = s2_ref.shape[0]

    @pl.when(mm == 0)
    def _prep():
        @pl.when(l == 0)
        def _init():
            s_scr[...] = s2_ref[...]

        s = s_scr[...]
        m = jnp.sum(s, axis=0, keepdims=True) * (1.0 / n)
        t = s - m
        g = jnp.maximum(jnp.max(jnp.abs(t), axis=0, keepdims=True),
                        1e-30) * (1.0 / _F8MAX)
        m_scr[...] = m
        g_scr[...] = g
        t8_scr[...] = (t * (1.0 / g)).astype(_F8)

    t8 = t8_scr[...]
    acc = jnp.dot(a8h_ref[mm], t8[:k],
                  preferred_element_type=jnp.float32)
    acc += jnp.dot(a8c_ref[...], t8[k:],
                   preferred_element_type=jnp.float32)
    h = acc * g_scr[...] + rs_ref[...] * m_scr[...] + b_ref[0]

    @pl.when(l < 6)
    def _next_support():
        s_scr[pl.ds(mm * bm, bm), :] = jnp.dot(
            jnp.maximum(h, 0.0), w_ref[0], preferred_element_type=jnp.float32)

    @pl.when(l == 6)
    def _emit():
        o_ref[...] = h[:, :nclass]


@functools.partial(jax.jit, static_argnames=("bm1", "bm", "k"))
def _gcn8(x, adj, Ws, bs, bm1=400, bm=400, k=3600):
    n, _ = x.shape
    nhid = Ws[0].shape[1]
    nclass = Ws[7].shape[1]
    f32 = jnp.float32

    s1 = pl.pallas_call(
        _support0_kernel,
        out_shape=jax.ShapeDtypeStruct((n, nhid), jnp.bfloat16),
    )(x, Ws[0])

    m1 = n // bm1
    a8h, a8c, rs, s2 = pl.pallas_call(
        functools.partial(_layer1_kernel, k=k),
        grid=(m1,),
        in_specs=[
            pl.BlockSpec((bm1, n), lambda m: (m, 0)),
            pl.BlockSpec((n, nhid), lambda m: (0, 0)),
            pl.BlockSpec((1, nhid), lambda m: (0, 0)),
            pl.BlockSpec((nhid, nhid), lambda m: (0, 0)),
        ],
        out_specs=[
            pl.BlockSpec((1, bm1, k), lambda m: (m, 0, 0)),
            pl.BlockSpec((bm1, n - k), lambda m: (m, 0)),
            pl.BlockSpec((bm1, 1), lambda m: (m, 0)),
            pl.BlockSpec((bm1, nhid), lambda m: (m, 0)),
        ],
        out_shape=[
            jax.ShapeDtypeStruct((m1, bm1, k), _F8),
            jax.ShapeDtypeStruct((n, n - k), _F8),
            jax.ShapeDtypeStruct((n, 1), f32),
            jax.ShapeDtypeStruct((n, nhid), f32),
        ],
    )(adj, s1, bs[0].reshape(1, nhid).astype(f32), Ws[1])

    wst = jnp.stack(
        [Ws[i] for i in range(2, 7)]
        + [jnp.pad(Ws[7], ((0, 0), (0, nhid - nclass)))]
        + [jnp.zeros((nhid, nhid), f32)], axis=0)
    bst = jnp.stack(
        [bs[i].astype(f32) for i in range(1, 7)]
        + [jnp.pad(bs[7].astype(f32), (0, nhid - nclass))],
        axis=0).reshape(7, 1, nhid)

    mg = n // bm
    out = pl.pallas_call(
        functools.partial(_fused7_kernel, k=k, bm=bm, nclass=nclass),
        grid=(7, mg),
        in_specs=[
            pl.BlockSpec((m1, bm1, k), lambda l, mm: (0, 0, 0)),
            pl.BlockSpec((bm, n - k), lambda l, mm: (mm, 0)),
            pl.BlockSpec((bm, 1), lambda l, mm: (mm, 0)),
            pl.BlockSpec((n, nhid), lambda l, mm: (0, 0)),
            pl.BlockSpec((1, nhid, nhid), lambda l, mm: (l, 0, 0)),
            pl.BlockSpec((1, 1, nhid), lambda l, mm: (l, 0, 0)),
        ],
        out_specs=pl.BlockSpec((bm, nclass), lambda l, mm: (mm, 0)),
        out_shape=jax.ShapeDtypeStruct((n, nclass), f32),
        scratch_shapes=[
            pltpu.VMEM((n, nhid), f32),
            pltpu.VMEM((n, nhid), _F8),
            pltpu.VMEM((1, nhid), f32),
            pltpu.VMEM((1, nhid), f32),
        ],
    )(a8h, a8c, rs, s2, wst, bst)
    return out


def kernel(x, adj, W1, b1, W2, b2, W3, b3, W4, b4, W5, b5, W6, b6, W7, b7, W8, b8):
    Ws = (W1, W2, W3, W4, W5, W6, W7, W8)
    bs = (b1, b2, b3, b4, b5, b6, b7, b8)
    return _gcn8(x, adj, Ws, bs)


# speedup vs baseline: 2.1948x; 1.1207x over previous
import functools

import jax
import jax.numpy as jnp
from jax.experimental import pallas as pl
from jax.experimental.pallas import tpu as pltpu

_F8 = jnp.float8_e4m3fn
_F8MAX = 448.0


def _support0_kernel(x_ref, w_ref, o_ref):
    o_ref[...] = jnp.dot(
        x_ref[...], w_ref[...], preferred_element_type=jnp.float32
    ).astype(jnp.bfloat16)


def _layer1_kernel(adj_ref, s_ref, b_ref, wn_ref,
                   a8h_ref, a8c_ref, rs_ref, sn_ref, *, k):
    a = adj_ref[...]
    a8 = a.astype(_F8)
    bm = a8.shape[0]
    a8h_ref[...] = a8[:, :k].reshape(1, bm, k)
    a8c_ref[...] = a8[:, k:]
    rs_ref[...] = jnp.sum(a, axis=1, keepdims=True)
    h = jnp.dot(a.astype(jnp.bfloat16), s_ref[...],
                preferred_element_type=jnp.float32)
    h = jnp.maximum(h + b_ref[...], 0.0)
    sn_ref[...] = jnp.dot(h, wn_ref[...], preferred_element_type=jnp.float32)


def _fused7_kernel(a8h_ref, a8c_ref, rs_ref, s2_ref, w_ref, b_ref,
                   o_ref, s_scr, t8_scr, m_scr, g_scr, *, k, bm, nclass):
    l = pl.program_id(0)
    mm = pl.program_id(1)
    n = s2_ref.shape[0]

    @pl.when(mm == 0)
    def _prep():
        @pl.when(l == 0)
        def _init():
            s_scr[...] = s2_ref[...]

        s = s_scr[...]
        m = jnp.sum(s, axis=0, keepdims=True) * (1.0 / n)
        t = s - m
        g = jnp.maximum(jnp.max(jnp.abs(t), axis=0, keepdims=True),
                        1e-30) * (1.0 / _F8MAX)
        m_scr[...] = m
        g_scr[...] = g
        t8_scr[...] = (t * (1.0 / g)).astype(_F8)

    t8 = t8_scr[...]
    acc = jnp.dot(a8h_ref[mm], t8[:k],
                  preferred_element_type=jnp.float32)
    acc += jnp.dot(a8c_ref[...], t8[k:],
                   preferred_element_type=jnp.float32)
    h = acc * g_scr[...] + rs_ref[...] * m_scr[...] + b_ref[0]

    @pl.when(l < 6)
    def _next_support():
        s_scr[pl.ds(mm * bm, bm), :] = jnp.dot(
            jnp.maximum(h, 0.0), w_ref[0], preferred_element_type=jnp.float32)

    @pl.when(l == 6)
    def _emit():
        o_ref[...] = h[:, :nclass]


@functools.partial(jax.jit, static_argnames=("bm1", "bm", "k"))
def _gcn8(x, adj, Ws, bs, bm1=400, bm=1000, k=2400):
    n, _ = x.shape
    nhid = Ws[0].shape[1]
    nclass = Ws[7].shape[1]
    f32 = jnp.float32

    s1 = pl.pallas_call(
        _support0_kernel,
        out_shape=jax.ShapeDtypeStruct((n, nhid), jnp.bfloat16),
    )(x, Ws[0])

    m1 = n // bm1
    a8h, a8c, rs, s2 = pl.pallas_call(
        functools.partial(_layer1_kernel, k=k),
        grid=(m1,),
        in_specs=[
            pl.BlockSpec((bm1, n), lambda m: (m, 0)),
            pl.BlockSpec((n, nhid), lambda m: (0, 0)),
            pl.BlockSpec((1, nhid), lambda m: (0, 0)),
            pl.BlockSpec((nhid, nhid), lambda m: (0, 0)),
        ],
        out_specs=[
            pl.BlockSpec((1, bm1, k), lambda m: (m, 0, 0)),
            pl.BlockSpec((bm1, n - k), lambda m: (m, 0)),
            pl.BlockSpec((bm1, 1), lambda m: (m, 0)),
            pl.BlockSpec((bm1, nhid), lambda m: (m, 0)),
        ],
        out_shape=[
            jax.ShapeDtypeStruct((m1, bm1, k), _F8),
            jax.ShapeDtypeStruct((n, n - k), _F8),
            jax.ShapeDtypeStruct((n, 1), f32),
            jax.ShapeDtypeStruct((n, nhid), f32),
        ],
    )(adj, s1, bs[0].reshape(1, nhid).astype(f32), Ws[1])

    wst = jnp.stack(
        [Ws[i] for i in range(2, 7)]
        + [jnp.pad(Ws[7], ((0, 0), (0, nhid - nclass)))]
        + [jnp.zeros((nhid, nhid), f32)], axis=0)
    bst = jnp.stack(
        [bs[i].astype(f32) for i in range(1, 7)]
        + [jnp.pad(bs[7].astype(f32), (0, nhid - nclass))],
        axis=0).reshape(7, 1, nhid)

    mg = n // bm
    a8h = a8h.reshape(mg, bm, k)
    out = pl.pallas_call(
        functools.partial(_fused7_kernel, k=k, bm=bm, nclass=nclass),
        grid=(7, mg),
        in_specs=[
            pl.BlockSpec((mg, bm, k), lambda l, mm: (0, 0, 0)),
            pl.BlockSpec((bm, n - k), lambda l, mm: (mm, 0)),
            pl.BlockSpec((bm, 1), lambda l, mm: (mm, 0)),
            pl.BlockSpec((n, nhid), lambda l, mm: (0, 0)),
            pl.BlockSpec((1, nhid, nhid), lambda l, mm: (l, 0, 0)),
            pl.BlockSpec((1, 1, nhid), lambda l, mm: (l, 0, 0)),
        ],
        out_specs=pl.BlockSpec((bm, nclass), lambda l, mm: (mm, 0)),
        out_shape=jax.ShapeDtypeStruct((n, nclass), f32),
        scratch_shapes=[
            pltpu.VMEM((n, nhid), f32),
            pltpu.VMEM((n, nhid), _F8),
            pltpu.VMEM((1, nhid), f32),
            pltpu.VMEM((1, nhid), f32),
        ],
    )(a8h, a8c, rs, s2, wst, bst)
    return out


def kernel(x, adj, W1, b1, W2, b2, W3, b3, W4, b4, W5, b5, W6, b6, W7, b7, W8, b8):
    Ws = (W1, W2, W3, W4, W5, W6, W7, W8)
    bs = (b1, b2, b3, b4, b5, b6, b7, b8)
    return _gcn8(x, adj, Ws, bs)


# speedup vs baseline: 2.2704x; 1.0345x over previous
import functools

import jax
import jax.numpy as jnp
from jax.experimental import pallas as pl
from jax.experimental.pallas import tpu as pltpu

_F8 = jnp.float8_e4m3fn
_F8MAX = 448.0
_GMARGIN = 1.3


def _support0_kernel(x_ref, w_ref, o_ref):
    o_ref[...] = jnp.dot(
        x_ref[...], w_ref[...], preferred_element_type=jnp.float32
    ).astype(jnp.bfloat16)


def _quant_stats(sn, bm):
    mn = jnp.sum(sn, axis=0, keepdims=True) * (1.0 / bm)
    gn = jnp.maximum(jnp.max(jnp.abs(sn - mn), axis=0, keepdims=True),
                     1e-30) * (_GMARGIN / _F8MAX)
    return mn, gn


def _quantize(sn, mn, gn):
    return jnp.clip((sn - mn) * (1.0 / gn), -_F8MAX, _F8MAX).astype(_F8)


def _layer1_kernel(adj_ref, s_ref, b_ref, wn_ref,
                   a8h_ref, a8c_ref, rs_ref, t8_ref, m_ref, g_ref,
                   m_scr, g_scr, *, k):
    a = adj_ref[...]
    a8 = a.astype(_F8)
    bm = a8.shape[0]
    a8h_ref[...] = a8[:, :k].reshape(1, bm, k)
    a8c_ref[...] = a8[:, k:]
    rs_ref[...] = jnp.sum(a, axis=1, keepdims=True)
    h = jnp.dot(a.astype(jnp.bfloat16), s_ref[...],
                preferred_element_type=jnp.float32)
    h = jnp.maximum(h + b_ref[...], 0.0)
    sn = jnp.dot(h, wn_ref[...], preferred_element_type=jnp.float32)

    @pl.when(pl.program_id(0) == 0)
    def _stats():
        mn, gn = _quant_stats(sn, bm)
        m_scr[...] = mn
        g_scr[...] = gn

    mn = m_scr[...]
    gn = g_scr[...]
    t8_ref[...] = _quantize(sn, mn, gn)
    m_ref[...] = mn
    g_ref[...] = gn


def _fused7_kernel(a8h_ref, a8c_ref, rs_ref, t80_ref, m0_ref, g0_ref,
                   w_ref, b_ref, o_ref, s_scr, t8_scr, m_scr, g_scr,
                   *, k, bm, nclass):
    l = pl.program_id(0)
    mm = pl.program_id(1)
    par = jax.lax.rem(l, 2)
    nxt = 1 - par

    @pl.when(mm == 0)
    def _boundary():
        @pl.when(l == 0)
        def _init():
            t8_scr[...] = t80_ref[...]
            m_scr[0] = m0_ref[...]
            g_scr[0] = g0_ref[...]

        @pl.when(l > 0)
        def _convert():
            t8_scr[...] = _quantize(s_scr[...], m_scr[par], g_scr[par])

    t8 = t8_scr[...]
    acc = jnp.dot(a8h_ref[mm], t8[:k], preferred_element_type=jnp.float32)
    acc += jnp.dot(a8c_ref[...], t8[k:], preferred_element_type=jnp.float32)
    h = acc * g_scr[par] + rs_ref[...] * m_scr[par] + b_ref[0]

    @pl.when(l < 6)
    def _next_support():
        sn = jnp.dot(jnp.maximum(h, 0.0), w_ref[0],
                     preferred_element_type=jnp.float32)
        s_scr[pl.ds(mm * bm, bm), :] = sn

        @pl.when(mm == 0)
        def _stats():
            mn, gn = _quant_stats(sn, bm)
            m_scr[nxt] = mn
            g_scr[nxt] = gn

    @pl.when(l == 6)
    def _emit():
        o_ref[...] = h[:, :nclass]


@functools.partial(jax.jit, static_argnames=("bm1", "bm", "k"))
def _gcn8(x, adj, Ws, bs, bm1=400, bm=1000, k=2400):
    n, _ = x.shape
    nhid = Ws[0].shape[1]
    nclass = Ws[7].shape[1]
    f32 = jnp.float32

    s1 = pl.pallas_call(
        _support0_kernel,
        out_shape=jax.ShapeDtypeStruct((n, nhid), jnp.bfloat16),
    )(x, Ws[0])

    m1 = n // bm1
    a8h, a8c, rs, t80, m0, g0 = pl.pallas_call(
        functools.partial(_layer1_kernel, k=k),
        grid=(m1,),
        in_specs=[
            pl.BlockSpec((bm1, n), lambda m: (m, 0)),
            pl.BlockSpec((n, nhid), lambda m: (0, 0)),
            pl.BlockSpec((1, nhid), lambda m: (0, 0)),
            pl.BlockSpec((nhid, nhid), lambda m: (0, 0)),
        ],
        out_specs=[
            pl.BlockSpec((1, bm1, k), lambda m: (m, 0, 0)),
            pl.BlockSpec((bm1, n - k), lambda m: (m, 0)),
            pl.BlockSpec((bm1, 1), lambda m: (m, 0)),
            pl.BlockSpec((bm1, nhid), lambda m: (m, 0)),
            pl.BlockSpec((1, nhid), lambda m: (0, 0)),
            pl.BlockSpec((1, nhid), lambda m: (0, 0)),
        ],
        out_shape=[
            jax.ShapeDtypeStruct((m1, bm1, k), _F8),
            jax.ShapeDtypeStruct((n, n - k), _F8),
            jax.ShapeDtypeStruct((n, 1), f32),
            jax.ShapeDtypeStruct((n, nhid), _F8),
            jax.ShapeDtypeStruct((1, nhid), f32),
            jax.ShapeDtypeStruct((1, nhid), f32),
        ],
        scratch_shapes=[
            pltpu.VMEM((1, nhid), f32),
            pltpu.VMEM((1, nhid), f32),
        ],
    )(adj, s1, bs[0].reshape(1, nhid).astype(f32), Ws[1])

    wst = jnp.stack(
        [Ws[i] for i in range(2, 7)]
        + [jnp.pad(Ws[7], ((0, 0), (0, nhid - nclass)))]
        + [jnp.zeros((nhid, nhid), f32)], axis=0)
    bst = jnp.stack(
        [bs[i].astype(f32) for i in range(1, 7)]
        + [jnp.pad(bs[7].astype(f32), (0, nhid - nclass))],
        axis=0).reshape(7, 1, nhid)

    mg = n // bm
    a8h = a8h.reshape(mg, bm, k)
    out = pl.pallas_call(
        functools.partial(_fused7_kernel, k=k, bm=bm, nclass=nclass),
        grid=(7, mg),
        in_specs=[
            pl.BlockSpec((mg, bm, k), lambda l, mm: (0, 0, 0)),
            pl.BlockSpec((bm, n - k), lambda l, mm: (mm, 0)),
            pl.BlockSpec((bm, 1), lambda l, mm: (mm, 0)),
            pl.BlockSpec((n, nhid), lambda l, mm: (0, 0)),
            pl.BlockSpec((1, nhid), lambda l, mm: (0, 0)),
            pl.BlockSpec((1, nhid), lambda l, mm: (0, 0)),
            pl.BlockSpec((1, nhid, nhid), lambda l, mm: (l, 0, 0)),
            pl.BlockSpec((1, 1, nhid), lambda l, mm: (l, 0, 0)),
        ],
        out_specs=pl.BlockSpec((bm, nclass), lambda l, mm: (mm, 0)),
        out_shape=jax.ShapeDtypeStruct((n, nclass), f32),
        scratch_shapes=[
            pltpu.VMEM((n, nhid), f32),
            pltpu.VMEM((n, nhid), _F8),
            pltpu.VMEM((2, 1, nhid), f32),
            pltpu.VMEM((2, 1, nhid), f32),
        ],
    )(a8h, a8c, rs, t80, m0, g0, wst, bst)
    return out


def kernel(x, adj, W1, b1, W2, b2, W3, b3, W4, b4, W5, b5, W6, b6, W7, b7, W8, b8):
    Ws = (W1, W2, W3, W4, W5, W6, W7, W8)
    bs = (b1, b2, b3, b4, b5, b6, b7, b8)
    return _gcn8(x, adj, Ws, bs)


# speedup vs baseline: 2.3277x; 1.0252x over previous
import functools

import jax
import jax.numpy as jnp
from jax.experimental import pallas as pl
from jax.experimental.pallas import tpu as pltpu

_F8 = jnp.float8_e4m3fn
_F8MAX = 448.0
_GMARGIN = 1.3


def _support0_kernel(x_ref, w_ref, o_ref):
    o_ref[...] = jnp.dot(
        x_ref[...], w_ref[...], preferred_element_type=jnp.float32
    ).astype(jnp.bfloat16)


def _quant_stats(sn, bm):
    mn = jnp.sum(sn, axis=0, keepdims=True) * (1.0 / bm)
    gn = jnp.maximum(jnp.max(jnp.abs(sn - mn), axis=0, keepdims=True),
                     1e-30) * (_GMARGIN / _F8MAX)
    return mn, gn


def _quantize(sn, mn, gn):
    return jnp.clip((sn - mn) * (1.0 / gn), -_F8MAX, _F8MAX).astype(_F8)


def _layer1_kernel(adj_ref, s_ref, b_ref, wn_ref,
                   a8h_ref, a8c_ref, rs_ref, t8_ref, m_ref, g_ref,
                   m_scr, g_scr, *, k):
    a = adj_ref[...]
    a8 = a.astype(_F8)
    bm = a8.shape[0]
    a8h_ref[...] = a8[:, :k].reshape(1, bm, k)
    a8c_ref[...] = a8[:, k:]
    rs_ref[...] = jnp.sum(a, axis=1, keepdims=True)
    h = jnp.dot(a.astype(jnp.bfloat16), s_ref[...],
                preferred_element_type=jnp.float32)
    h = jnp.maximum(h + b_ref[...], 0.0)
    sn = jnp.dot(h, wn_ref[...], preferred_element_type=jnp.float32)

    @pl.when(pl.program_id(0) == 0)
    def _stats():
        mn, gn = _quant_stats(sn, bm)
        m_scr[...] = mn
        g_scr[...] = gn

    mn = m_scr[...]
    gn = g_scr[...]
    t8_ref[...] = _quantize(sn, mn, gn)
    m_ref[...] = mn
    g_ref[...] = gn


def _fused7_kernel(a8h_ref, a8c_ref, rs_ref, t80_ref, m0_ref, g0_ref,
                   w_ref, b_ref, o_ref, s_scr, t8_scr, m_scr, g_scr,
                   *, k, bm, nclass):
    l = pl.program_id(0)
    mm = pl.program_id(1)
    par = jax.lax.rem(l, 2)
    nxt = 1 - par

    @pl.when(mm == 0)
    def _boundary():
        @pl.when(l == 0)
        def _init():
            t8_scr[...] = t80_ref[...]
            m_scr[0] = m0_ref[...]
            g_scr[0] = g0_ref[...]

        @pl.when(l > 0)
        def _convert():
            t8_scr[...] = _quantize(s_scr[...], m_scr[par], g_scr[par])

    t8 = t8_scr[...]
    acc = jnp.dot(a8h_ref[mm], t8[:k], preferred_element_type=jnp.float32)
    acc += jnp.dot(a8c_ref[...], t8[k:], preferred_element_type=jnp.float32)
    rsb = rs_ref[pl.ds(mm * bm, bm), :]
    h = acc * g_scr[par] + rsb * m_scr[par] + b_ref[l]

    @pl.when(l < 6)
    def _next_support():
        sn = jnp.dot(jnp.maximum(h, 0.0), w_ref[l],
                     preferred_element_type=jnp.float32)
        s_scr[pl.ds(mm * bm, bm), :] = sn

        @pl.when(mm == 0)
        def _stats():
            mn, gn = _quant_stats(sn, bm)
            m_scr[nxt] = mn
            g_scr[nxt] = gn

    @pl.when(l == 6)
    def _emit():
        o_ref[...] = h[:, :nclass]


@functools.partial(jax.jit, static_argnames=("bm1", "bm", "k"))
def _gcn8(x, adj, Ws, bs, bm1=400, bm=1000, k=2400):
    n, _ = x.shape
    nhid = Ws[0].shape[1]
    nclass = Ws[7].shape[1]
    f32 = jnp.float32

    s1 = pl.pallas_call(
        _support0_kernel,
        out_shape=jax.ShapeDtypeStruct((n, nhid), jnp.bfloat16),
    )(x, Ws[0])

    m1 = n // bm1
    a8h, a8c, rs, t80, m0, g0 = pl.pallas_call(
        functools.partial(_layer1_kernel, k=k),
        grid=(m1,),
        in_specs=[
            pl.BlockSpec((bm1, n), lambda m: (m, 0)),
            pl.BlockSpec((n, nhid), lambda m: (0, 0)),
            pl.BlockSpec((1, nhid), lambda m: (0, 0)),
            pl.BlockSpec((nhid, nhid), lambda m: (0, 0)),
        ],
        out_specs=[
            pl.BlockSpec((1, bm1, k), lambda m: (m, 0, 0)),
            pl.BlockSpec((bm1, n - k), lambda m: (m, 0)),
            pl.BlockSpec((bm1, 1), lambda m: (m, 0)),
            pl.BlockSpec((bm1, nhid), lambda m: (m, 0)),
            pl.BlockSpec((1, nhid), lambda m: (0, 0)),
            pl.BlockSpec((1, nhid), lambda m: (0, 0)),
        ],
        out_shape=[
            jax.ShapeDtypeStruct((m1, bm1, k), _F8),
            jax.ShapeDtypeStruct((n, n - k), _F8),
            jax.ShapeDtypeStruct((n, 1), f32),
            jax.ShapeDtypeStruct((n, nhid), _F8),
            jax.ShapeDtypeStruct((1, nhid), f32),
            jax.ShapeDtypeStruct((1, nhid), f32),
        ],
        scratch_shapes=[
            pltpu.VMEM((1, nhid), f32),
            pltpu.VMEM((1, nhid), f32),
        ],
    )(adj, s1, bs[0].reshape(1, nhid).astype(f32), Ws[1])

    wst = jnp.stack(
        [Ws[i] for i in range(2, 7)]
        + [jnp.pad(Ws[7], ((0, 0), (0, nhid - nclass)))]
        + [jnp.zeros((nhid, nhid), f32)], axis=0)
    bst = jnp.stack(
        [bs[i].astype(f32) for i in range(1, 7)]
        + [jnp.pad(bs[7].astype(f32), (0, nhid - nclass))],
        axis=0).reshape(7, 1, nhid)

    mg = n // bm
    a8h = a8h.reshape(mg, bm, k)
    out = pl.pallas_call(
        functools.partial(_fused7_kernel, k=k, bm=bm, nclass=nclass),
        grid=(7, mg),
        in_specs=[
            pl.BlockSpec((mg, bm, k), lambda l, mm: (0, 0, 0)),
            pl.BlockSpec((bm, n - k), lambda l, mm: (mm, 0)),
            pl.BlockSpec((n, 1), lambda l, mm: (0, 0)),
            pl.BlockSpec((n, nhid), lambda l, mm: (0, 0)),
            pl.BlockSpec((1, nhid), lambda l, mm: (0, 0)),
            pl.BlockSpec((1, nhid), lambda l, mm: (0, 0)),
            pl.BlockSpec((7, nhid, nhid), lambda l, mm: (0, 0, 0)),
            pl.BlockSpec((7, 1, nhid), lambda l, mm: (0, 0, 0)),
        ],
        out_specs=pl.BlockSpec(
            (bm, nclass),
            lambda l, mm: (jnp.where(l == 6, mm, 0), 0)),
        out_shape=jax.ShapeDtypeStruct((n, nclass), f32),
        scratch_shapes=[
            pltpu.VMEM((n, nhid), f32),
            pltpu.VMEM((n, nhid), _F8),
            pltpu.VMEM((2, 1, nhid), f32),
            pltpu.VMEM((2, 1, nhid), f32),
        ],
    )(a8h, a8c, rs, t80, m0, g0, wst, bst)
    return out


def kernel(x, adj, W1, b1, W2, b2, W3, b3, W4, b4, W5, b5, W6, b6, W7, b7, W8, b8):
    Ws = (W1, W2, W3, W4, W5, W6, W7, W8)
    bs = (b1, b2, b3, b4, b5, b6, b7, b8)
    return _gcn8(x, adj, Ws, bs)


# speedup vs baseline: 2.3477x; 1.0086x over previous
import functools

import jax
import jax.numpy as jnp
from jax.experimental import pallas as pl
from jax.experimental.pallas import tpu as pltpu

_F8 = jnp.float8_e4m3fn
_F8MAX = 448.0
_GMARGIN = 1.3


def _support0_kernel(x_ref, w_ref, o_ref):
    o_ref[...] = jnp.dot(
        x_ref[...], w_ref[...], preferred_element_type=jnp.float32
    ).astype(jnp.bfloat16)


def _quant_stats(sn, bm):
    mn = jnp.sum(sn, axis=0, keepdims=True) * (1.0 / bm)
    gn = jnp.maximum(jnp.max(jnp.abs(sn - mn), axis=0, keepdims=True),
                     1e-30) * (_GMARGIN / _F8MAX)
    return mn, gn


def _quantize(sn, mn, gn):
    return jnp.clip((sn - mn) * (1.0 / gn), -_F8MAX, _F8MAX).astype(_F8)


def _layer1_kernel(adj_ref, s_ref, b_ref, wn_ref,
                   a8h_ref, a8c_ref, rs_ref, t8_ref, m_ref, g_ref,
                   m_scr, g_scr, *, k):
    a = adj_ref[...]
    a8 = a.astype(_F8)
    bm = a8.shape[0]
    a8h_ref[...] = a8[:, :k].reshape(1, bm, k)
    a8c_ref[...] = a8[:, k:]
    rs_ref[...] = jnp.sum(a, axis=1, keepdims=True)
    h = jnp.dot(a.astype(jnp.bfloat16), s_ref[...],
                preferred_element_type=jnp.float32)
    h = jnp.maximum(h + b_ref[...], 0.0)
    sn = jnp.dot(h, wn_ref[...], preferred_element_type=jnp.float32)

    @pl.when(pl.program_id(0) == 0)
    def _stats():
        mn, gn = _quant_stats(sn, bm)
        m_scr[...] = mn
        g_scr[...] = gn

    mn = m_scr[...]
    gn = g_scr[...]
    t8_ref[...] = _quantize(sn, mn, gn)
    m_ref[...] = mn
    g_ref[...] = gn


def _fused7_kernel(a8h_ref, a8c_ref, rs_ref, t80_ref, m0_ref, g0_ref,
                   w_ref, b_ref, o_ref, s_scr, t8_scr, m_scr, g_scr,
                   *, k, bm, nclass):
    l = pl.program_id(0)
    mm = pl.program_id(1)
    par = jax.lax.rem(l, 2)
    nxt = 1 - par

    @pl.when(mm == 0)
    def _boundary():
        @pl.when(l == 0)
        def _init():
            t8_scr[...] = t80_ref[...]
            m_scr[0] = m0_ref[...]
            g_scr[0] = g0_ref[...]

        @pl.when(l > 0)
        def _convert():
            t8_scr[...] = _quantize(s_scr[...], m_scr[par], g_scr[par])

    t8 = t8_scr[...]
    acc = jnp.dot(a8h_ref[mm], t8[:k], preferred_element_type=jnp.float32)
    acc += jnp.dot(a8c_ref[...], t8[k:], preferred_element_type=jnp.float32)
    rsb = rs_ref[pl.ds(mm * bm, bm), :]
    h = acc * g_scr[par] + rsb * m_scr[par] + b_ref[l]

    @pl.when(l < 6)
    def _next_support():
        sn = jnp.dot(jnp.maximum(h, 0.0), w_ref[l],
                     preferred_element_type=jnp.float32)
        s_scr[pl.ds(mm * bm, bm), :] = sn

        @pl.when(mm == 0)
        def _stats():
            mn, gn = _quant_stats(sn, bm)
            m_scr[nxt] = mn
            g_scr[nxt] = gn

    @pl.when(l == 6)
    def _emit():
        o_ref[...] = h[:, :nclass]


@functools.partial(jax.jit, static_argnames=("bm1", "bm", "k"))
def _gcn8(x, adj, Ws, bs, bm1=400, bm=1000, k=2688):
    n, _ = x.shape
    nhid = Ws[0].shape[1]
    nclass = Ws[7].shape[1]
    f32 = jnp.float32

    s1 = pl.pallas_call(
        _support0_kernel,
        out_shape=jax.ShapeDtypeStruct((n, nhid), jnp.bfloat16),
    )(x, Ws[0])

    m1 = n // bm1
    a8h, a8c, rs, t80, m0, g0 = pl.pallas_call(
        functools.partial(_layer1_kernel, k=k),
        grid=(m1,),
        in_specs=[
            pl.BlockSpec((bm1, n), lambda m: (m, 0)),
            pl.BlockSpec((n, nhid), lambda m: (0, 0)),
            pl.BlockSpec((1, nhid), lambda m: (0, 0)),
            pl.BlockSpec((nhid, nhid), lambda m: (0, 0)),
        ],
        out_specs=[
            pl.BlockSpec((1, bm1, k), lambda m: (m, 0, 0)),
            pl.BlockSpec((bm1, n - k), lambda m: (m, 0)),
            pl.BlockSpec((bm1, 1), lambda m: (m, 0)),
            pl.BlockSpec((bm1, nhid), lambda m: (m, 0)),
            pl.BlockSpec((1, nhid), lambda m: (0, 0)),
            pl.BlockSpec((1, nhid), lambda m: (0, 0)),
        ],
        out_shape=[
            jax.ShapeDtypeStruct((m1, bm1, k), _F8),
            jax.ShapeDtypeStruct((n, n - k), _F8),
            jax.ShapeDtypeStruct((n, 1), f32),
            jax.ShapeDtypeStruct((n, nhid), _F8),
            jax.ShapeDtypeStruct((1, nhid), f32),
            jax.ShapeDtypeStruct((1, nhid), f32),
        ],
        scratch_shapes=[
            pltpu.VMEM((1, nhid), f32),
            pltpu.VMEM((1, nhid), f32),
        ],
    )(adj, s1, bs[0].reshape(1, nhid).astype(f32), Ws[1])

    wst = jnp.stack(
        [Ws[i] for i in range(2, 7)]
        + [jnp.pad(Ws[7], ((0, 0), (0, nhid - nclass)))]
        + [jnp.zeros((nhid, nhid), f32)], axis=0)
    bst = jnp.stack(
        [bs[i].astype(f32) for i in range(1, 7)]
        + [jnp.pad(bs[7].astype(f32), (0, nhid - nclass))],
        axis=0).reshape(7, 1, nhid)

    mg = n // bm
    a8h = a8h.reshape(mg, bm, k)
    out = pl.pallas_call(
        functools.partial(_fused7_kernel, k=k, bm=bm, nclass=nclass),
        grid=(7, mg),
        in_specs=[
            pl.BlockSpec((mg, bm, k), lambda l, mm: (0, 0, 0)),
            pl.BlockSpec((bm, n - k), lambda l, mm: (mm, 0)),
            pl.BlockSpec((n, 1), lambda l, mm: (0, 0)),
            pl.BlockSpec((n, nhid), lambda l, mm: (0, 0)),
            pl.BlockSpec((1, nhid), lambda l, mm: (0, 0)),
            pl.BlockSpec((1, nhid), lambda l, mm: (0, 0)),
            pl.BlockSpec((7, nhid, nhid), lambda l, mm: (0, 0, 0)),
            pl.BlockSpec((7, 1, nhid), lambda l, mm: (0, 0, 0)),
        ],
        out_specs=pl.BlockSpec(
            (bm, nclass),
            lambda l, mm: (jnp.where(l == 6, mm, 0), 0)),
        out_shape=jax.ShapeDtypeStruct((n, nclass), f32),
        scratch_shapes=[
            pltpu.VMEM((n, nhid), f32),
            pltpu.VMEM((n, nhid), _F8),
            pltpu.VMEM((2, 1, nhid), f32),
            pltpu.VMEM((2, 1, nhid), f32),
        ],
    )(a8h, a8c, rs, t80, m0, g0, wst, bst)
    return out


def kernel(x, adj, W1, b1, W2, b2, W3, b3, W4, b4, W5, b5, W6, b6, W7, b7, W8, b8):
    Ws = (W1, W2, W3, W4, W5, W6, W7, W8)
    bs = (b1, b2, b3, b4, b5, b6, b7, b8)
    return _gcn8(x, adj, Ws, bs)
